# baseline (speedup 1.0000x reference)
import functools

import jax
import jax.numpy as jnp
from jax import lax
from jax.experimental import pallas as pl
from jax.experimental.pallas import tpu as pltpu
from jax.experimental.pallas import tpu_sc as plsc

_EPS = 1e-07
_ALPHA = 0.5

_NC = 2
_NS = 16
_NW = _NC * _NS
_L = 16


def _spline_tables(cps0):
    cps = jnp.concatenate([cps0, cps0[0:1, :]], axis=0)
    l01 = jnp.sqrt(jnp.sum(jnp.power(cps[0, :] - cps[1, :], 2)) + _EPS)
    l_last = jnp.sqrt(jnp.sum(jnp.power(cps[-1, :] - cps[-2, :], 2)) + _EPS)
    first = cps[0, :] - l01 / l_last * (cps[-1, :] - cps[-2, :])
    last = cps[-1, :] + l_last / l01 * (cps[1, :] - cps[0, :])
    aux = jnp.concatenate([first[None, :], cps, last[None, :]], axis=0)
    d = jnp.power(jnp.sum(jnp.power(aux[1:] - aux[:-1], 2), axis=-1),
                  _ALPHA / 2.0)
    tk = jnp.concatenate([jnp.zeros(1, dtype=jnp.float32), jnp.cumsum(d)])
    return aux, tk


def _poly_shift(p):
    return jnp.concatenate([jnp.zeros((1,), jnp.float32), p[:3]])


def _poly_lerp(p, q, ta, tb):
    r = 1.0 / (tb - ta)
    return (tb * p - _poly_shift(p) + _poly_shift(q) - ta * q) * r


def _segment_cubic(aux, tk, s, d):
    t = [tk[s - 1 + i] for i in range(4)]
    a = [jnp.concatenate([aux[s - 1 + i, d][None],
                          jnp.zeros((3,), jnp.float32)]) for i in range(4)]
    x01 = _poly_lerp(a[0], a[1], t[0], t[1])
    x12 = _poly_lerp(a[1], a[2], t[1], t[2])
    x23 = _poly_lerp(a[2], a[3], t[2], t[3])
    x012 = _poly_lerp(x01, x12, t[0], t[2])
    x123 = _poly_lerp(x12, x23, t[1], t[3])
    return _poly_lerp(x012, x123, t[1], t[2])


def _make_sc_eval(n, chunk):
    nvec = chunk // _L
    mesh = plsc.VectorSubcoreMesh(core_axis_name="c", subcore_axis_name="s",
                                  num_cores=_NC, num_subcores=_NS)

    @functools.partial(
        pl.kernel,
        out_type=jax.ShapeDtypeStruct((2 * n,), jnp.float32),
        mesh=mesh,
        compiler_params=pltpu.CompilerParams(needs_layout_passes=False),
        scratch_types=[
            pltpu.VMEM((chunk,), jnp.float32),
            pltpu.VMEM((2 * chunk,), jnp.float32),
            pltpu.VMEM((17 * _L,), jnp.float32),
        ],
    )
    def spline_eval(t_hbm, c_hbm, out_hbm, tbuf, obuf, cbuf):
        wid = lax.axis_index("s") * _NC + lax.axis_index("c")
        base = jnp.minimum(wid * chunk, n - chunk)
        pltpu.sync_copy(t_hbm.at[pl.ds(base, chunk)], tbuf)
        pltpu.sync_copy(c_hbm, cbuf)

        c = [cbuf[pl.ds(_L * k, _L)] for k in range(17)]
        tk2 = c[0]
        iot2 = lax.iota(jnp.int32, _L) * 2

        def step(j, carry):
            tv = tbuf[pl.ds(j * _L, _L)]
            m = tv >= tk2
            idx = iot2 + j * (2 * _L)
            for d in range(2):
                o1, o2 = 1 + 4 * d, 9 + 4 * d
                cc = [jnp.where(m, c[o2 + i], c[o1 + i]) for i in range(4)]
                p = ((cc[3] * tv + cc[2]) * tv + cc[1]) * tv + cc[0]
                plsc.store_scatter(obuf, [idx + d], p)
            return carry

        lax.fori_loop(0, nvec, step, 0, unroll=4)
        pltpu.sync_copy(obuf, out_hbm.at[pl.ds(2 * base, 2 * chunk)])

    return spline_eval


def kernel(t, cps):
    n = t.shape[0]
    aux, tk = _spline_tables(cps)

    rows = [tk[2]]
    for s in (1, 2):
        for d in (0, 1):
            rows += list(_segment_cubic(aux, tk, s, d))
    consts = jnp.stack(rows).astype(jnp.float32)
    cvec = jnp.broadcast_to(consts[:, None], (17, _L)).reshape(-1)

    assert n % 8 == 0
    gran = _NW * _L
    chunk = ((n + gran - 1) // gran) * _L
    flat = _make_sc_eval(n, chunk)(t, cvec)
    return flat.reshape(n, 2)


# speedup vs baseline: 1.0000x; 1.0000x over previous
import functools

import jax
import jax.numpy as jnp
from jax import lax
from jax.experimental import pallas as pl
from jax.experimental.pallas import tpu as pltpu
from jax.experimental.pallas import tpu_sc as plsc

_EPS = 1e-07
_ALPHA = 0.5

_NC = 2
_NS = 16
_NW = _NC * _NS
_L = 16


def _spline_tables(cps0):
    cps = jnp.concatenate([cps0, cps0[0:1, :]], axis=0)
    l01 = jnp.sqrt(jnp.sum(jnp.power(cps[0, :] - cps[1, :], 2)) + _EPS)
    l_last = jnp.sqrt(jnp.sum(jnp.power(cps[-1, :] - cps[-2, :], 2)) + _EPS)
    first = cps[0, :] - l01 / l_last * (cps[-1, :] - cps[-2, :])
    last = cps[-1, :] + l_last / l01 * (cps[1, :] - cps[0, :])
    aux = jnp.concatenate([first[None, :], cps, last[None, :]], axis=0)
    d = jnp.power(jnp.sum(jnp.power(aux[1:] - aux[:-1], 2), axis=-1),
                  _ALPHA / 2.0)
    tk = jnp.concatenate([jnp.zeros(1, dtype=jnp.float32), jnp.cumsum(d)])
    return aux, tk


def _poly_shift(p):
    return jnp.concatenate([jnp.zeros((1,), jnp.float32), p[:3]])


def _poly_lerp(p, q, ta, tb):
    r = 1.0 / (tb - ta)
    return (tb * p - _poly_shift(p) + _poly_shift(q) - ta * q) * r


def _segment_cubic(aux, tk, s, d):
    t = [tk[s - 1 + i] for i in range(4)]
    a = [jnp.concatenate([aux[s - 1 + i, d][None],
                          jnp.zeros((3,), jnp.float32)]) for i in range(4)]
    x01 = _poly_lerp(a[0], a[1], t[0], t[1])
    x12 = _poly_lerp(a[1], a[2], t[1], t[2])
    x23 = _poly_lerp(a[2], a[3], t[2], t[3])
    x012 = _poly_lerp(x01, x12, t[0], t[2])
    x123 = _poly_lerp(x12, x23, t[1], t[3])
    return _poly_lerp(x012, x123, t[1], t[2])


def _make_sc_eval(n, chunk):
    nvec = chunk // _L
    mesh = plsc.VectorSubcoreMesh(core_axis_name="c", subcore_axis_name="s",
                                  num_cores=_NC, num_subcores=_NS)

    @functools.partial(
        pl.kernel,
        out_type=jax.ShapeDtypeStruct((2 * n,), jnp.float32),
        mesh=mesh,
        compiler_params=pltpu.CompilerParams(needs_layout_passes=False),
        scratch_types=[
            pltpu.VMEM((chunk,), jnp.float32),
            pltpu.VMEM((2 * chunk,), jnp.float32),
            pltpu.VMEM((17 * _L,), jnp.float32),
        ],
    )
    def spline_eval(t_hbm, c_hbm, out_hbm, tbuf, obuf, cbuf):
        wid = lax.axis_index("s") * _NC + lax.axis_index("c")
        base = jnp.minimum(wid * chunk, n - chunk)
        pltpu.sync_copy(t_hbm.at[pl.ds(base, chunk)], tbuf)
        pltpu.sync_copy(c_hbm, cbuf)

        c = [cbuf[pl.ds(_L * k, _L)] for k in range(17)]
        tk2 = c[0]
        iot2 = lax.iota(jnp.int32, _L) * 2

        def step(j, carry):
            tv = tbuf[pl.ds(j * _L, _L)]
            m = tv >= tk2
            idx = iot2 + j * (2 * _L)
            for d in range(2):
                o1, o2 = 1 + 4 * d, 9 + 4 * d
                cc = [jnp.where(m, c[o2 + i], c[o1 + i]) for i in range(4)]
                p = ((cc[3] * tv + cc[2]) * tv + cc[1]) * tv + cc[0]
                plsc.store_scatter(obuf, [idx + d], p)
            return carry

        lax.fori_loop(0, nvec, step, 0)
        pltpu.sync_copy(obuf, out_hbm.at[pl.ds(2 * base, 2 * chunk)])

    return spline_eval


def kernel(t, cps):
    n = t.shape[0]
    aux, tk = _spline_tables(cps)

    rows = [tk[2]]
    for s in (1, 2):
        for d in (0, 1):
            rows += list(_segment_cubic(aux, tk, s, d))
    consts = jnp.stack(rows).astype(jnp.float32)
    cvec = jnp.broadcast_to(consts[:, None], (17, _L)).reshape(-1)

    assert n % 8 == 0
    gran = _NW * _L
    chunk = ((n + gran - 1) // gran) * _L
    flat = _make_sc_eval(n, chunk)(t, cvec)
    return flat.reshape(n, 2)
